# Initial kernel scaffold; baseline (speedup 1.0000x reference)
import jax
import jax.numpy as jnp
from jax.experimental import pallas as pl


def kernel(item_table, user_table, w_ih, w_hh, b_ih, b_hh, w_out, b_out, item_id, user_ids, word_embeddings):
    raise NotImplementedError("write your pallas kernel here")


# speedup vs baseline: 2.3809x; 2.3809x over previous
import jax
import jax.numpy as jnp
from jax.experimental import pallas as pl
from jax.experimental.pallas import tpu as pltpu


def _fused_kernel(x_ref, w_ih_ref, w_hh_ref, b_ih_ref, b_hh_ref,
                  item_ref, user_ref, w_out_ref, b_out_ref, out_ref,
                  *, seq_len):
    x = x_ref[...]
    xr = (jnp.dot(x, w_ih_ref[0], preferred_element_type=jnp.float32)
          + (b_ih_ref[0] + b_hh_ref[0]))
    xz = (jnp.dot(x, w_ih_ref[1], preferred_element_type=jnp.float32)
          + (b_ih_ref[1] + b_hh_ref[1]))
    xn = (jnp.dot(x, w_ih_ref[2], preferred_element_type=jnp.float32)
          + b_ih_ref[2])

    ur = w_hh_ref[0]
    uz = w_hh_ref[1]
    un = w_hh_ref[2]
    b_hn = b_hh_ref[2]

    H = ur.shape[0]
    h = jnp.zeros((1, H), jnp.float32)
    h_sum = jnp.zeros((1, H), jnp.float32)

    for t in range(seq_len):
        hb = h.astype(jnp.bfloat16)
        hr = jnp.dot(hb, ur, preferred_element_type=jnp.float32)
        hz = jnp.dot(hb, uz, preferred_element_type=jnp.float32)
        hn = jnp.dot(hb, un, preferred_element_type=jnp.float32)
        r = jax.nn.sigmoid(xr[t:t + 1, :] + hr)
        z = jax.nn.sigmoid(xz[t:t + 1, :] + hz)
        n = jnp.tanh(xn[t:t + 1, :] + r * (hn + b_hn))
        h = n + z * (h - n)
        h_sum = h_sum + h

    mean_h = h_sum * (1.0 / float(seq_len))

    scale = item_ref[...] * mean_h
    mul = user_ref[...] * scale
    logits = (jnp.dot(mul, w_out_ref[...],
                      preferred_element_type=jnp.float32)
              + b_out_ref[...])
    m = jnp.max(logits, axis=-1, keepdims=True)
    e = jnp.exp(logits - m)
    out_ref[...] = e / jnp.sum(e, axis=-1, keepdims=True)


def kernel(item_table, user_table, w_ih, w_hh, b_ih, b_hh, w_out, b_out,
           item_id, user_ids, word_embeddings):
    seq_len, batch, emb_dim = word_embeddings.shape
    hidden = w_hh.shape[-1]
    rating_range = w_out.shape[-1]
    assert batch == 1 and hidden == emb_dim

    x = word_embeddings.reshape(seq_len, emb_dim).astype(jnp.bfloat16)
    w_ih_b = w_ih.astype(jnp.bfloat16)
    w_hh_b = w_hh.astype(jnp.bfloat16)

    item_emb = item_table[item_id][None, :]
    user_emb = user_table[jnp.asarray(user_ids)]
    num_users = user_emb.shape[0]

    import functools
    kern = functools.partial(_fused_kernel, seq_len=seq_len)
    return pl.pallas_call(
        kern,
        out_shape=jax.ShapeDtypeStruct((num_users, rating_range),
                                       jnp.float32),
        compiler_params=pltpu.CompilerParams(
            dimension_semantics=()),
    )(x, w_ih_b, w_hh_b, b_ih, b_hh, item_emb, user_emb, w_out, b_out)
